# Initial kernel scaffold; baseline (speedup 1.0000x reference)
import jax
import jax.numpy as jnp
from jax.experimental import pallas as pl


def kernel(pc, feature, coarse, W1, b1, W2, b2, Wr1, br1, Wr2, br2, Wr3, br3):
    raise NotImplementedError("write your pallas kernel here")


# speedup vs baseline: 5.9734x; 5.9734x over previous
import jax
import jax.numpy as jnp
from jax.experimental import pallas as pl

f32 = jnp.float32
bf16 = jnp.bfloat16
B, N, K = 16, 2048, 16
M = 512
NKEEP = 1536
RT = 256


def _bq(x):
    return x.astype(bf16).astype(f32)


def _lf_kernel(c_ref, ct_ref, xc_ref, xr_ref, o_ref):
    c = c_ref[0]
    ct = ct_ref[0]
    dot = jnp.dot(_bq(c), _bq(ct), preferred_element_type=f32)
    pd = (-xr_ref[0] + jnp.float32(2.0) * dot) - xc_ref[0]
    t = jax.lax.bitcast_convert_type(pd, jnp.int32)
    t = jnp.where(t < 0, jnp.int32(0x7FFFFFFF) ^ t, t)
    iota = jax.lax.broadcasted_iota(jnp.int32, (RT, N), 1)
    cx = ct[0:1, :]; cy = ct[1:2, :]; cz = ct[2:3, :]
    sx = c[:, 0:1]; sy = c[:, 1:2]; sz = c[:, 2:3]
    NEG = jnp.float32(-3.4e38)
    mx = jnp.full((RT, 1), NEG); my = jnp.full((RT, 1), NEG)
    mz = jnp.full((RT, 1), NEG)

    def body(k, carry):
        t_, mx_, my_, mz_ = carry
        m = jnp.max(t_, axis=1, keepdims=True)
        sel = jnp.min(jnp.where(t_ == m, iota, N), axis=1, keepdims=True)
        onehot = iota == sel
        nx = jnp.max(jnp.where(onehot, cx, NEG), axis=1, keepdims=True)
        ny = jnp.max(jnp.where(onehot, cy, NEG), axis=1, keepdims=True)
        nz = jnp.max(jnp.where(onehot, cz, NEG), axis=1, keepdims=True)
        mx_ = jnp.maximum(mx_, nx - sx)
        my_ = jnp.maximum(my_, ny - sy)
        mz_ = jnp.maximum(mz_, nz - sz)
        t_ = jnp.where(onehot, jnp.int32(-2147483647 - 1), t_)
        return t_, mx_, my_, mz_

    t, mx, my, mz = jax.lax.fori_loop(0, K, body, (t, mx, my, mz))
    o_ref[0] = jnp.concatenate([mx, my, mz], axis=1)


def _lf_pallas(c):
    ct = jnp.swapaxes(c, 1, 2)
    xx = jnp.sum(ct ** 2, axis=1)
    return pl.pallas_call(
        _lf_kernel,
        grid=(B, N // RT),
        in_specs=[pl.BlockSpec((1, RT, 3), lambda b, r: (b, r, 0)),
                  pl.BlockSpec((1, 3, N), lambda b, r: (b, 0, 0)),
                  pl.BlockSpec((1, RT, 1), lambda b, r: (b, r, 0)),
                  pl.BlockSpec((1, 1, N), lambda b, r: (b, 0, 0))],
        out_specs=pl.BlockSpec((1, RT, 3), lambda b, r: (b, r, 0)),
        out_shape=jax.ShapeDtypeStruct((B, N, 3), f32),
    )(c, ct, xx[:, :, None], xx[:, None, :])


def _fps_kernel(px_ref, py_ref, pz_ref, ox_ref, oy_ref, oz_ref):
    px, py, pz = px_ref[...], py_ref[...], pz_ref[...]
    iota = jax.lax.broadcasted_iota(jnp.int32, (B, N), 1)
    iota_m = jax.lax.broadcasted_iota(jnp.int32, (B, M), 1)
    mind0 = jnp.full((B, N), 1e10, dtype=f32)
    lx0, ly0, lz0 = px[:, 0:1], py[:, 0:1], pz[:, 0:1]
    ox0 = jnp.where(iota_m == 0, lx0, 0.0)
    oy0 = jnp.where(iota_m == 0, ly0, 0.0)
    oz0 = jnp.where(iota_m == 0, lz0, 0.0)

    def body(i, carry):
        mind, lx, ly, lz, ox, oy, oz = carry
        dx = px - lx; dy = py - ly; dz = pz - lz
        dsq = dx * dx + dy * dy + dz * dz
        mind = jnp.minimum(mind, dsq)
        m = jnp.max(mind, axis=1, keepdims=True)
        sel = jnp.min(jnp.where(mind == m, iota, N), axis=1, keepdims=True)
        onehot = iota == sel
        lx = jnp.sum(jnp.where(onehot, px, 0.0), axis=1, keepdims=True)
        ly = jnp.sum(jnp.where(onehot, py, 0.0), axis=1, keepdims=True)
        lz = jnp.sum(jnp.where(onehot, pz, 0.0), axis=1, keepdims=True)
        put = iota_m == i
        ox = jnp.where(put, lx, ox)
        oy = jnp.where(put, ly, oy)
        oz = jnp.where(put, lz, oz)
        return (mind, lx, ly, lz, ox, oy, oz)

    carry = (mind0, lx0, ly0, lz0, ox0, oy0, oz0)
    carry = jax.lax.fori_loop(1, M, body, carry)
    _, _, _, _, ox, oy, oz = carry
    ox_ref[...] = ox; oy_ref[...] = oy; oz_ref[...] = oz


def _fps_pallas(p):
    px, py, pz = p[:, :, 0], p[:, :, 1], p[:, :, 2]
    ox, oy, oz = pl.pallas_call(
        _fps_kernel,
        out_shape=(jax.ShapeDtypeStruct((B, M), f32),
                   jax.ShapeDtypeStruct((B, M), f32),
                   jax.ShapeDtypeStruct((B, M), f32)),
    )(px, py, pz)
    return jnp.stack([ox, oy, oz], axis=-1)


def kernel(pc, feature, coarse, W1, b1, W2, b2, Wr1, br1, Wr2, br2, Wr3, br3):
    local_feature = _lf_pallas(coarse)

    gf = jnp.max(feature, axis=1)
    gf = jax.nn.gelu(gf @ W1 + b1) @ W2 + b2
    eg = jnp.broadcast_to(gf[:, None, :], (B, N, gf.shape[-1]))
    cf = jnp.concatenate([eg, coarse, local_feature], axis=-1)
    h = jax.nn.gelu(cf @ Wr1 + br1)
    h = jax.nn.gelu(h @ Wr2 + br2)
    confidence_score = jax.nn.sigmoid(h @ Wr3 + br3)

    idx = jnp.argsort(-confidence_score, axis=1)
    top = jnp.broadcast_to(idx[:, :NKEEP], (B, NKEEP, 3))
    coarse_best = jnp.take_along_axis(coarse, top, axis=1)

    coarse_inp = _fps_pallas(pc)

    coarse_final = jnp.concatenate([coarse_best, coarse_inp], axis=1)
    return (coarse, confidence_score, coarse_final)


# speedup vs baseline: 6.5393x; 1.0947x over previous
import jax
import jax.numpy as jnp
from jax.experimental import pallas as pl

f32 = jnp.float32
bf16 = jnp.bfloat16
B, N, K = 16, 2048, 16
M = 512
NKEEP = 1536
RT = 256


def _bq(x):
    return x.astype(bf16).astype(f32)


def _lf_kernel(c_ref, ct_ref, xc_ref, xr_ref, o_ref):
    c = c_ref[0]
    ct = ct_ref[0]
    dot = jnp.dot(_bq(c), _bq(ct), preferred_element_type=f32)
    pd = (-xr_ref[0] + jnp.float32(2.0) * dot) - xc_ref[0]
    t = jax.lax.bitcast_convert_type(pd, jnp.int32)
    t = jnp.where(t < 0, jnp.int32(0x7FFFFFFF) ^ t, t)
    iota = jax.lax.broadcasted_iota(jnp.int32, (RT, N), 1)
    cx = ct[0:1, :]; cy = ct[1:2, :]; cz = ct[2:3, :]
    sx = c[:, 0:1]; sy = c[:, 1:2]; sz = c[:, 2:3]
    NEG = jnp.float32(-3.4e38)

    def body(k, carry):
        t_, selmask = carry
        m = jnp.max(t_, axis=1, keepdims=True)
        sel = jnp.min(jnp.where(t_ == m, iota, N), axis=1, keepdims=True)
        onehot = iota == sel
        selmask = jnp.where(onehot, jnp.float32(1.0), selmask)
        t_ = jnp.where(onehot, jnp.int32(-2147483647 - 1), t_)
        return t_, selmask

    selmask0 = jnp.zeros((RT, N), dtype=f32)
    t, selmask = jax.lax.fori_loop(0, K, body, (t, selmask0))
    keep = selmask > 0
    mx = jnp.max(jnp.where(keep, cx - sx, NEG), axis=1, keepdims=True)
    my = jnp.max(jnp.where(keep, cy - sy, NEG), axis=1, keepdims=True)
    mz = jnp.max(jnp.where(keep, cz - sz, NEG), axis=1, keepdims=True)
    o_ref[0] = jnp.concatenate([mx, my, mz], axis=1)


def _lf_pallas(c):
    ct = jnp.swapaxes(c, 1, 2)
    xx = jnp.sum(ct ** 2, axis=1)
    return pl.pallas_call(
        _lf_kernel,
        grid=(B, N // RT),
        in_specs=[pl.BlockSpec((1, RT, 3), lambda b, r: (b, r, 0)),
                  pl.BlockSpec((1, 3, N), lambda b, r: (b, 0, 0)),
                  pl.BlockSpec((1, RT, 1), lambda b, r: (b, r, 0)),
                  pl.BlockSpec((1, 1, N), lambda b, r: (b, 0, 0))],
        out_specs=pl.BlockSpec((1, RT, 3), lambda b, r: (b, r, 0)),
        out_shape=jax.ShapeDtypeStruct((B, N, 3), f32),
    )(c, ct, xx[:, :, None], xx[:, None, :])


def _fps_kernel(px_ref, py_ref, pz_ref, ox_ref, oy_ref, oz_ref):
    px, py, pz = px_ref[...], py_ref[...], pz_ref[...]
    iota = jax.lax.broadcasted_iota(jnp.int32, (B, N), 1)
    iota_m = jax.lax.broadcasted_iota(jnp.int32, (B, M), 1)
    mind0 = jnp.full((B, N), 1e10, dtype=f32)
    lx0, ly0, lz0 = px[:, 0:1], py[:, 0:1], pz[:, 0:1]
    ox0 = jnp.where(iota_m == 0, lx0, 0.0)
    oy0 = jnp.where(iota_m == 0, ly0, 0.0)
    oz0 = jnp.where(iota_m == 0, lz0, 0.0)

    def body(i, carry):
        mind, lx, ly, lz, ox, oy, oz = carry
        dx = px - lx; dy = py - ly; dz = pz - lz
        dsq = dx * dx + dy * dy + dz * dz
        mind = jnp.minimum(mind, dsq)
        m = jnp.max(mind, axis=1, keepdims=True)
        sel = jnp.min(jnp.where(mind == m, iota, N), axis=1, keepdims=True)
        onehot = iota == sel
        lx = jnp.sum(jnp.where(onehot, px, 0.0), axis=1, keepdims=True)
        ly = jnp.sum(jnp.where(onehot, py, 0.0), axis=1, keepdims=True)
        lz = jnp.sum(jnp.where(onehot, pz, 0.0), axis=1, keepdims=True)
        put = iota_m == i
        ox = jnp.where(put, lx, ox)
        oy = jnp.where(put, ly, oy)
        oz = jnp.where(put, lz, oz)
        return (mind, lx, ly, lz, ox, oy, oz)

    carry = (mind0, lx0, ly0, lz0, ox0, oy0, oz0)
    carry = jax.lax.fori_loop(1, M, body, carry)
    _, _, _, _, ox, oy, oz = carry
    ox_ref[...] = ox; oy_ref[...] = oy; oz_ref[...] = oz


def _fps_pallas(p):
    px, py, pz = p[:, :, 0], p[:, :, 1], p[:, :, 2]
    ox, oy, oz = pl.pallas_call(
        _fps_kernel,
        out_shape=(jax.ShapeDtypeStruct((B, M), f32),
                   jax.ShapeDtypeStruct((B, M), f32),
                   jax.ShapeDtypeStruct((B, M), f32)),
    )(px, py, pz)
    return jnp.stack([ox, oy, oz], axis=-1)


def kernel(pc, feature, coarse, W1, b1, W2, b2, Wr1, br1, Wr2, br2, Wr3, br3):
    local_feature = _lf_pallas(coarse)

    gf = jnp.max(feature, axis=1)
    gf = jax.nn.gelu(gf @ W1 + b1) @ W2 + b2
    eg = jnp.broadcast_to(gf[:, None, :], (B, N, gf.shape[-1]))
    cf = jnp.concatenate([eg, coarse, local_feature], axis=-1)
    h = jax.nn.gelu(cf @ Wr1 + br1)
    h = jax.nn.gelu(h @ Wr2 + br2)
    confidence_score = jax.nn.sigmoid(h @ Wr3 + br3)

    idx = jnp.argsort(-confidence_score, axis=1)
    top = jnp.broadcast_to(idx[:, :NKEEP], (B, NKEEP, 3))
    coarse_best = jnp.take_along_axis(coarse, top, axis=1)

    coarse_inp = _fps_pallas(pc)

    coarse_final = jnp.concatenate([coarse_best, coarse_inp], axis=1)
    return (coarse, confidence_score, coarse_final)
